# baseline (speedup 1.0000x reference)
import jax
import jax.numpy as jnp
from jax.experimental import pallas as pl
from jax.experimental.pallas import tpu as pltpu

_RADIUS = 5
_H = 128
_W = 128
_IN = _H - 2 * _RADIUS

_DELTAS = [
    (dx, dy)
    for dx in range(-_RADIUS, _RADIUS + 1)
    for dy in range(-_RADIUS, _RADIUS + 1)
    if dx * dx + dy * dy < _RADIUS * _RADIUS and dx + dy != 0
]


def _loss_kernel(df_ref, tg_ref, out_ref, tp_ref, r0_ref, r1_ref, rt_ref):
    r = _RADIUS
    tp_ref[...] = jnp.where(tg_ref[...] > 0, jnp.float32(1.0), jnp.float32(0.0))

    for j in range(2 * _RADIUS - 1):
        ys = j + 1
        r0_ref[j] = df_ref[:, 0, ys:ys + _IN, :]
        r1_ref[j] = df_ref[:, 1, ys:ys + _IN, :]
        rt_ref[j] = tp_ref[:, ys:ys + _IN, :]

    f0c = [r0_ref[r - 1, 2 * h:2 * h + 2, :, r:r + _IN] for h in range(2)]
    f1c = [r1_ref[r - 1, 2 * h:2 * h + 2, :, r:r + _IN] for h in range(2)]
    tcf = [rt_ref[r - 1, 2 * h:2 * h + 2, :, r:r + _IN] for h in range(2)]

    accf = jnp.zeros((_IN, _IN), jnp.float32)
    accb = jnp.zeros((_IN, _IN), jnp.float32)
    accc = jnp.zeros((_IN, _IN), jnp.float32)
    for dx, dy in _DELTAS:
        j = r + dy - 1
        xs = r + dx
        for h in range(2):
            bs = 2 * h
            d0 = r0_ref[j, bs:bs + 2, :, xs:xs + _IN] - f0c[h]
            d1 = r1_ref[j, bs:bs + 2, :, xs:xs + _IN] - f1c[h]
            fgf = tcf[h] * rt_ref[j, bs:bs + 2, :, xs:xs + _IN]
            ab = jnp.abs(d0 - jnp.float32(dy)) + jnp.abs(d1 - jnp.float32(dx))
            s = d0 + d1
            accf = accf + jnp.sum(fgf * ab, axis=0)
            accb = accb + jnp.sum(s - fgf * s, axis=0)
            accc = accc + jnp.sum(fgf, axis=0)

    fg_sum = jnp.sum(accf)
    bg_sum = jnp.sum(accb)
    fg_cnt = jnp.sum(accc)
    total = jnp.float32(len(_DELTAS) * _IN * _IN * tg_ref.shape[0])
    bg_cnt = total - fg_cnt
    loss = (fg_sum / jnp.maximum(fg_cnt, 1.0)
            + bg_sum / jnp.maximum(bg_cnt, 1.0))
    out_ref[:, :] = loss[None, None]


def kernel(df, bd, targets):
    del bd
    B = df.shape[0]
    out = pl.pallas_call(
        _loss_kernel,
        out_shape=jax.ShapeDtypeStruct((1, 1), jnp.float32),
        scratch_shapes=[
            pltpu.VMEM((B, _H, _W), jnp.float32),
            pltpu.VMEM((2 * _RADIUS - 1, B, _IN, _W), jnp.float32),
            pltpu.VMEM((2 * _RADIUS - 1, B, _IN, _W), jnp.float32),
            pltpu.VMEM((2 * _RADIUS - 1, B, _IN, _W), jnp.float32),
        ],
    )(df, targets)
    return out[0, 0]


# speedup vs baseline: 1.2604x; 1.2604x over previous
import jax
import jax.numpy as jnp
from jax.experimental import pallas as pl
from jax.experimental.pallas import tpu as pltpu

_RADIUS = 5
_H = 128
_W = 128
_IN = _H - 2 * _RADIUS

_DELTAS = [
    (dx, dy)
    for dx in range(-_RADIUS, _RADIUS + 1)
    for dy in range(-_RADIUS, _RADIUS + 1)
    if dx * dx + dy * dy < _RADIUS * _RADIUS and dx + dy != 0
]


def _loss_kernel(df_ref, tg_ref, out_ref, r0_ref, r1_ref, rt_ref):
    r = _RADIUS

    for j in range(2 * _RADIUS - 1):
        ys = j + 1
        r0_ref[j] = df_ref[:, 0, ys:ys + _IN, :].astype(jnp.bfloat16)
        r1_ref[j] = df_ref[:, 1, ys:ys + _IN, :].astype(jnp.bfloat16)
        rt_ref[j] = jnp.where(tg_ref[:, ys:ys + _IN, :] > 0,
                              jnp.float32(1.0),
                              jnp.float32(0.0)).astype(jnp.bfloat16)

    f0c = r0_ref[r - 1, :, :, r:r + _IN]
    f1c = r1_ref[r - 1, :, :, r:r + _IN]
    tcf = rt_ref[r - 1, :, :, r:r + _IN]

    accf = jnp.zeros((_IN, _IN), jnp.float32)
    accb = jnp.zeros((_IN, _IN), jnp.float32)
    accc = jnp.zeros((_IN, _IN), jnp.float32)
    for dx, dy in _DELTAS:
        j = r + dy - 1
        xs = r + dx
        d0 = r0_ref[j, :, :, xs:xs + _IN] - f0c
        d1 = r1_ref[j, :, :, xs:xs + _IN] - f1c
        fgf = tcf * rt_ref[j, :, :, xs:xs + _IN]
        ab = (jnp.abs(d0 - jnp.bfloat16(dy))
              + jnp.abs(d1 - jnp.bfloat16(dx)))
        s = d0 + d1
        accf = accf + jnp.sum(fgf * ab, axis=0).astype(jnp.float32)
        accb = accb + jnp.sum(s - fgf * s, axis=0).astype(jnp.float32)
        accc = accc + jnp.sum(fgf, axis=0).astype(jnp.float32)

    fg_sum = jnp.sum(accf)
    bg_sum = jnp.sum(accb)
    fg_cnt = jnp.sum(accc)
    total = jnp.float32(len(_DELTAS) * _IN * _IN * tg_ref.shape[0])
    bg_cnt = total - fg_cnt
    loss = (fg_sum / jnp.maximum(fg_cnt, 1.0)
            + bg_sum / jnp.maximum(bg_cnt, 1.0))
    out_ref[:, :] = loss[None, None]


def kernel(df, bd, targets):
    del bd
    B = df.shape[0]
    out = pl.pallas_call(
        _loss_kernel,
        out_shape=jax.ShapeDtypeStruct((1, 1), jnp.float32),
        scratch_shapes=[
            pltpu.VMEM((2 * _RADIUS - 1, B, _IN, _W), jnp.bfloat16),
            pltpu.VMEM((2 * _RADIUS - 1, B, _IN, _W), jnp.bfloat16),
            pltpu.VMEM((2 * _RADIUS - 1, B, _IN, _W), jnp.bfloat16),
        ],
    )(df, targets)
    return out[0, 0]


# speedup vs baseline: 1.3596x; 1.0787x over previous
import jax
import jax.numpy as jnp
from jax.experimental import pallas as pl
from jax.experimental.pallas import tpu as pltpu

_RADIUS = 5
_H = 128
_W = 128
_IN = _H - 2 * _RADIUS

_DELTAS = [
    (dx, dy)
    for dx in range(-_RADIUS, _RADIUS + 1)
    for dy in range(-_RADIUS, _RADIUS + 1)
    if dx * dx + dy * dy < _RADIUS * _RADIUS and dx + dy != 0
]


def _loss_kernel(df_ref, tg_ref, out_ref, r0_ref, r1_ref, rt_ref):
    r = _RADIUS

    for j in range(2 * _RADIUS - 1):
        ys = j + 1
        r0_ref[j] = df_ref[:, 0, ys:ys + _IN, :]
        r1_ref[j] = df_ref[:, 1, ys:ys + _IN, :]
        rt_ref[j] = jnp.where(tg_ref[:, ys:ys + _IN, :] > 0,
                              jnp.float32(1.0), jnp.float32(0.0))

    f0c = r0_ref[r - 1, :, :, r:r + _IN]
    f1c = r1_ref[r - 1, :, :, r:r + _IN]
    tcf = rt_ref[r - 1, :, :, r:r + _IN]

    accf = jnp.zeros((_IN, _IN), jnp.float32)
    accb = jnp.zeros((_IN, _IN), jnp.float32)
    accc = jnp.zeros((_IN, _IN), jnp.float32)
    for dx, dy in _DELTAS:
        j = r + dy - 1
        xs = r + dx
        d0 = r0_ref[j, :, :, xs:xs + _IN] - f0c
        d1 = r1_ref[j, :, :, xs:xs + _IN] - f1c
        fgf = tcf * rt_ref[j, :, :, xs:xs + _IN]
        ab = jnp.abs(d0 - jnp.float32(dy)) + jnp.abs(d1 - jnp.float32(dx))
        s = d0 + d1
        accf = accf + jnp.sum(fgf * ab, axis=0)
        accb = accb + jnp.sum(s - fgf * s, axis=0)
        accc = accc + jnp.sum(fgf, axis=0)

    fg_sum = jnp.sum(accf)
    bg_sum = jnp.sum(accb)
    fg_cnt = jnp.sum(accc)
    total = jnp.float32(len(_DELTAS) * _IN * _IN * tg_ref.shape[0])
    bg_cnt = total - fg_cnt
    loss = (fg_sum / jnp.maximum(fg_cnt, 1.0)
            + bg_sum / jnp.maximum(bg_cnt, 1.0))
    out_ref[:, :] = loss[None, None]


def kernel(df, bd, targets):
    del bd
    B = df.shape[0]
    out = pl.pallas_call(
        _loss_kernel,
        out_shape=jax.ShapeDtypeStruct((1, 1), jnp.float32),
        scratch_shapes=[
            pltpu.VMEM((2 * _RADIUS - 1, B, _IN, _W), jnp.float32),
            pltpu.VMEM((2 * _RADIUS - 1, B, _IN, _W), jnp.float32),
            pltpu.VMEM((2 * _RADIUS - 1, B, _IN, _W), jnp.float32),
        ],
    )(df, targets)
    return out.reshape(())


# speedup vs baseline: 2.0495x; 1.5075x over previous
import jax
import jax.numpy as jnp
from jax.experimental import pallas as pl
from jax.experimental.pallas import tpu as pltpu

_RADIUS = 5
_H = 128
_W = 128
_IN = _H - 2 * _RADIUS

_DELTAS = [
    (dx, dy)
    for dx in range(-_RADIUS, _RADIUS + 1)
    for dy in range(-_RADIUS, _RADIUS + 1)
    if dx * dx + dy * dy < _RADIUS * _RADIUS and dx + dy != 0
]


def _loss_kernel(df_ref, tg_ref, out_ref, r0_ref, r1_ref, rt_ref):
    r = _RADIUS

    for j in range(2 * _RADIUS - 1):
        ys = j + 1
        r0_ref[j] = df_ref[:, 0, ys:ys + _IN, :]
        r1_ref[j] = df_ref[:, 1, ys:ys + _IN, :]
        rt_ref[j] = jnp.where(tg_ref[:, ys:ys + _IN, :] > 0,
                              jnp.float32(1.0), jnp.float32(0.0))

    f0c = r0_ref[r - 1]
    f1c = r1_ref[r - 1]
    col = jax.lax.broadcasted_iota(jnp.int32, (_IN, _W), 1)
    vmask = jnp.where((col >= r) & (col < r + _IN),
                      jnp.float32(1.0), jnp.float32(0.0))
    tcf = rt_ref[r - 1] * vmask

    accf = jnp.zeros((_IN, _W), jnp.float32)
    accb = jnp.zeros((_IN, _W), jnp.float32)
    accc = jnp.zeros((_IN, _W), jnp.float32)
    def _rolled(ref, j, dx):
        v = ref[j]
        return v if dx == 0 else jnp.roll(v, -dx, axis=-1)

    for dx, dy in _DELTAS:
        j = r + dy - 1
        d0 = _rolled(r0_ref, j, dx) - f0c
        d1 = _rolled(r1_ref, j, dx) - f1c
        fgf = tcf * _rolled(rt_ref, j, dx)
        ab = jnp.abs(d0 - jnp.float32(dy)) + jnp.abs(d1 - jnp.float32(dx))
        s = d0 + d1
        accf = accf + jnp.sum(fgf * ab, axis=0)
        accb = accb + jnp.sum((vmask - fgf) * s, axis=0)
        accc = accc + jnp.sum(fgf, axis=0)

    fg_sum = jnp.sum(accf)
    bg_sum = jnp.sum(accb)
    fg_cnt = jnp.sum(accc)
    total = jnp.float32(len(_DELTAS) * _IN * _IN * tg_ref.shape[0])
    bg_cnt = total - fg_cnt
    loss = (fg_sum / jnp.maximum(fg_cnt, 1.0)
            + bg_sum / jnp.maximum(bg_cnt, 1.0))
    out_ref[:, :] = loss[None, None]


def kernel(df, bd, targets):
    del bd
    B = df.shape[0]
    out = pl.pallas_call(
        _loss_kernel,
        out_shape=jax.ShapeDtypeStruct((1, 1), jnp.float32),
        scratch_shapes=[
            pltpu.VMEM((2 * _RADIUS - 1, B, _IN, _W), jnp.float32),
            pltpu.VMEM((2 * _RADIUS - 1, B, _IN, _W), jnp.float32),
            pltpu.VMEM((2 * _RADIUS - 1, B, _IN, _W), jnp.float32),
        ],
    )(df, targets)
    return out.reshape(())
